# Initial kernel scaffold; baseline (speedup 1.0000x reference)
import jax
import jax.numpy as jnp
from jax.experimental import pallas as pl


def kernel(x):
    raise NotImplementedError("write your pallas kernel here")


# speedup vs baseline: 5.5243x; 5.5243x over previous
import jax
import jax.numpy as jnp
import numpy as np
from jax.experimental import pallas as pl

_ROWS = 8192
_COLS = 2048
_CHUNK = 128
_NCHUNK = _COLS // _CHUNK
_BLOCK_ROWS = 512


def _cumsum_kernel(x_ref, tri_ref, pre_ref, o_ref):
    x = x_ref[...]
    b = x.shape[0]
    x3 = x.reshape(b, _NCHUNK, _CHUNK)
    inner = jax.lax.dot_general(
        x3, tri_ref[...],
        dimension_numbers=(((2,), (0,)), ((), ())),
        preferred_element_type=jnp.float32,
    )
    totals = inner[:, :, _CHUNK - 1]
    offs = jax.lax.dot_general(
        totals, pre_ref[...],
        dimension_numbers=(((1,), (0,)), ((), ())),
        preferred_element_type=jnp.float32,
    )
    out = inner + offs[:, :, None]
    o_ref[...] = out.reshape(b, _COLS)


def _make_consts():
    tri = np.triu(np.ones((_CHUNK, _CHUNK), np.float32))
    pre = np.triu(np.ones((_NCHUNK, _NCHUNK), np.float32), 1)
    return jnp.asarray(tri), jnp.asarray(pre)


@jax.jit
def kernel(x):
    tri, pre = _make_consts()
    grid = (_ROWS // _BLOCK_ROWS,)
    return pl.pallas_call(
        _cumsum_kernel,
        grid=grid,
        in_specs=[
            pl.BlockSpec((_BLOCK_ROWS, _COLS), lambda i: (i, 0)),
            pl.BlockSpec((_CHUNK, _CHUNK), lambda i: (0, 0)),
            pl.BlockSpec((_NCHUNK, _NCHUNK), lambda i: (0, 0)),
        ],
        out_specs=pl.BlockSpec((_BLOCK_ROWS, _COLS), lambda i: (i, 0)),
        out_shape=jax.ShapeDtypeStruct((_ROWS, _COLS), jnp.float32),
    )(x, tri, pre)
